# Initial kernel scaffold; baseline (speedup 1.0000x reference)
import jax
import jax.numpy as jnp
from jax.experimental import pallas as pl


def kernel(x, edge_index, W1s, W1d, a1, b1, gamma, beta, W2s, W2d, a2, b2):
    raise NotImplementedError("write your pallas kernel here")


# speedup vs baseline: 3.9035x; 3.9035x over previous
import jax
import jax.numpy as jnp
from jax import lax
from jax.experimental import pallas as pl
from jax.experimental.pallas import tpu as pltpu
from jax.experimental.pallas import tpu_sc as plsc

N = 10000
E = 320000
D = 128
H1 = 64
H2 = 32

NC = 2
NS = 16
NW = NC * NS
L = 16

EPW = E // NW
CH = 80
NCHUNK = EPW // CH
RPT = 624
TAIL = N - NS * RPT


def _mesh():
    return plsc.VectorSubcoreMesh(core_axis_name="c", subcore_axis_name="s")


def _make_score_kernel(H):

    def body(xs_hbm, xd_hbm, src_hbm, dst_hbm, a_hbm,
             ex_hbm, denomp_hbm,
             xs_sp, xd_sp, denom_sp,
             src_v, dst_v, gx_v, gd_v, ex_v, a_v, zero_v, sem):
        c = lax.axis_index("c")
        s = lax.axis_index("s")
        wid = s * NC + c

        r0 = s * RPT
        pltpu.sync_copy(xs_hbm.at[pl.ds(r0, RPT)], xs_sp.at[pl.ds(r0, RPT)])
        pltpu.sync_copy(xd_hbm.at[pl.ds(r0, RPT)], xd_sp.at[pl.ds(r0, RPT)])

        def zb(i, _):
            zero_v[pl.ds(i * L, L)] = jnp.zeros((L,), jnp.float32)
            return 0
        lax.fori_loop(0, RPT // L, zb, 0, unroll=8)
        pltpu.sync_copy(zero_v, denom_sp.at[pl.ds(r0, RPT)])

        @pl.when(s == NS - 1)
        def _():
            pltpu.sync_copy(xs_hbm.at[pl.ds(NS * RPT, TAIL)],
                            xs_sp.at[pl.ds(NS * RPT, TAIL)])
            pltpu.sync_copy(xd_hbm.at[pl.ds(NS * RPT, TAIL)],
                            xd_sp.at[pl.ds(NS * RPT, TAIL)])
            pltpu.sync_copy(zero_v.at[pl.ds(0, TAIL)],
                            denom_sp.at[pl.ds(NS * RPT, TAIL)])

        pltpu.sync_copy(a_hbm, a_v)
        plsc.subcore_barrier()

        lanes = lax.iota(jnp.int32, L)

        def chunk_body(ci, _):
            base = wid * EPW + ci * CH
            pltpu.sync_copy(src_hbm.at[pl.ds(base, CH)], src_v)
            pltpu.sync_copy(dst_hbm.at[pl.ds(base, CH)], dst_v)
            pltpu.async_copy(xs_sp.at[src_v], gx_v, sem).wait()
            pltpu.async_copy(xd_sp.at[dst_v], gd_v, sem).wait()

            def hbody(h, es):
                colh = jnp.full((L,), 0, jnp.int32) + h
                ab = plsc.load_gather(a_v, [colh])
                out = []
                for g in range(CH // L):
                    row = lanes + (g * L)
                    vx = plsc.load_gather(gx_v, [row, colh])
                    vd = plsc.load_gather(gd_v, [row, colh])
                    m = vx + vd
                    m = jnp.maximum(m, 0.2 * m)
                    out.append(es[g] + m * ab)
                return tuple(out)
            es = lax.fori_loop(
                0, H, hbody,
                tuple(jnp.zeros((L,), jnp.float32) for _ in range(CH // L)),
                unroll=8)
            for g in range(CH // L):
                ex_v[pl.ds(g * L, L)] = jnp.exp(es[g])

            pltpu.sync_copy(ex_v, ex_hbm.at[pl.ds(base, CH)])
            pltpu.sync_copy(ex_v, denom_sp.at[dst_v], add=True)
            return 0
        lax.fori_loop(0, NCHUNK, chunk_body, 0)

        plsc.subcore_barrier()
        pltpu.sync_copy(denom_sp.at[pl.ds(r0, RPT)],
                        denomp_hbm.at[pl.ds(c * N + r0, RPT)])

        @pl.when(s == NS - 1)
        def _():
            pltpu.sync_copy(denom_sp.at[pl.ds(NS * RPT, TAIL)],
                            denomp_hbm.at[pl.ds(c * N + NS * RPT, TAIL)])

    return pl.kernel(
        body,
        out_type=(jax.ShapeDtypeStruct((E,), jnp.float32),
                  jax.ShapeDtypeStruct((NC * N,), jnp.float32)),
        mesh=_mesh(),
        compiler_params=pltpu.CompilerParams(use_tc_tiling_on_sc=False, needs_layout_passes=False),
        scratch_types=[
            pltpu.VMEM_SHARED((N, H), jnp.float32),
            pltpu.VMEM_SHARED((N, H), jnp.float32),
            pltpu.VMEM_SHARED((N,), jnp.float32),
            pltpu.VMEM((CH,), jnp.int32),
            pltpu.VMEM((CH,), jnp.int32),
            pltpu.VMEM((CH, H), jnp.float32),
            pltpu.VMEM((CH, H), jnp.float32),
            pltpu.VMEM((CH,), jnp.float32),
            pltpu.VMEM((H,), jnp.float32),
            pltpu.VMEM((RPT,), jnp.float32),
            pltpu.SemaphoreType.DMA,
        ],
    )


def _make_aggr_kernel(H):

    def body(xs_hbm, src_hbm, dst_hbm, ex_hbm, denomp_hbm,
             outp_hbm,
             xs_sp, out_sp,
             src_v, dst_v, gx_v, wb_v, ex_v, d0_v, d1_v, zrow_v, sem):
        c = lax.axis_index("c")
        s = lax.axis_index("s")
        wid = s * NC + c

        r0 = s * RPT
        pltpu.sync_copy(xs_hbm.at[pl.ds(r0, RPT)], xs_sp.at[pl.ds(r0, RPT)])

        def zb(i, _):
            for j in range(H // L):
                zrow_v[i, pl.ds(j * L, L)] = jnp.zeros((L,), jnp.float32)
            return 0
        lax.fori_loop(0, RPT // 3, zb, 0, unroll=8)
        for q in range(3):
            pltpu.sync_copy(zrow_v, out_sp.at[pl.ds(r0 + q * (RPT // 3),
                                                    RPT // 3)])

        @pl.when(s == NS - 1)
        def _():
            pltpu.sync_copy(xs_hbm.at[pl.ds(NS * RPT, TAIL)],
                            xs_sp.at[pl.ds(NS * RPT, TAIL)])
            pltpu.sync_copy(zrow_v.at[pl.ds(0, TAIL)],
                            out_sp.at[pl.ds(NS * RPT, TAIL)])

        pltpu.sync_copy(denomp_hbm.at[pl.ds(0, N)], d0_v)
        pltpu.sync_copy(denomp_hbm.at[pl.ds(N, N)], d1_v)

        def addb(i, _):
            d0_v[pl.ds(i * L, L)] = (d0_v[pl.ds(i * L, L)]
                                     + d1_v[pl.ds(i * L, L)] + 1e-16)
            return 0
        lax.fori_loop(0, N // L, addb, 0, unroll=8)
        plsc.subcore_barrier()

        lanes = lax.iota(jnp.int32, L)

        def chunk_body(ci, _):
            base = wid * EPW + ci * CH
            pltpu.sync_copy(src_hbm.at[pl.ds(base, CH)], src_v)
            pltpu.sync_copy(dst_hbm.at[pl.ds(base, CH)], dst_v)
            pltpu.sync_copy(ex_hbm.at[pl.ds(base, CH)], ex_v)
            pltpu.async_copy(xs_sp.at[src_v], gx_v, sem).wait()

            for g in range(CH // L):
                row = lanes + (g * L)
                didx = dst_v[pl.ds(g * L, L)]
                dv = plsc.load_gather(d0_v, [didx])
                alpha = ex_v[pl.ds(g * L, L)] / dv

                def hbody(h, _):
                    colh = jnp.full((L,), 0, jnp.int32) + h
                    w = plsc.load_gather(gx_v, [row, colh]) * alpha
                    plsc.store_scatter(wb_v, [row, colh], w)
                    return 0
                lax.fori_loop(0, H, hbody, 0, unroll=16)

            pltpu.sync_copy(wb_v, out_sp.at[dst_v], add=True)
            return 0
        lax.fori_loop(0, NCHUNK, chunk_body, 0)

        plsc.subcore_barrier()
        pltpu.sync_copy(out_sp.at[pl.ds(r0, RPT)],
                        outp_hbm.at[c, pl.ds(r0, RPT)])

        @pl.when(s == NS - 1)
        def _():
            pltpu.sync_copy(out_sp.at[pl.ds(NS * RPT, TAIL)],
                            outp_hbm.at[c, pl.ds(NS * RPT, TAIL)])

    return pl.kernel(
        body,
        out_type=jax.ShapeDtypeStruct((NC, N, H), jnp.float32),
        mesh=_mesh(),
        compiler_params=pltpu.CompilerParams(use_tc_tiling_on_sc=False, needs_layout_passes=False),
        scratch_types=[
            pltpu.VMEM_SHARED((N, H), jnp.float32),
            pltpu.VMEM_SHARED((N, H), jnp.float32),
            pltpu.VMEM((CH,), jnp.int32),
            pltpu.VMEM((CH,), jnp.int32),
            pltpu.VMEM((CH, H), jnp.float32),
            pltpu.VMEM((CH, H), jnp.float32),
            pltpu.VMEM((CH,), jnp.float32),
            pltpu.VMEM((N,), jnp.float32),
            pltpu.VMEM((N,), jnp.float32),
            pltpu.VMEM((RPT // 3, H), jnp.float32),
            pltpu.SemaphoreType.DMA,
        ],
    )


def _proj_body(x_ref, ws_ref, wd_ref, xs_ref, xd_ref):
    x = x_ref[...]
    xs_ref[...] = jnp.dot(x, ws_ref[...], preferred_element_type=jnp.float32)
    xd_ref[...] = jnp.dot(x, wd_ref[...], preferred_element_type=jnp.float32)


def _proj(x, ws, wd, h):
    return pl.pallas_call(
        _proj_body,
        out_shape=(jax.ShapeDtypeStruct((N, h), jnp.float32),
                   jax.ShapeDtypeStruct((N, h), jnp.float32)),
    )(x, ws, wd)


def _mid_body(p_ref, b_ref, g_ref, be_ref, ws_ref, wd_ref, xs_ref, xd_ref):
    h = p_ref[0] + p_ref[1] + b_ref[...]
    mean = jnp.mean(h, axis=0)
    var = jnp.mean((h - mean) ** 2, axis=0)
    h = (h - mean) / jnp.sqrt(var + 1e-5) * g_ref[...] + be_ref[...]
    h = jnp.maximum(h, 0.0)
    xs_ref[...] = jnp.dot(h, ws_ref[...], preferred_element_type=jnp.float32)
    xd_ref[...] = jnp.dot(h, wd_ref[...], preferred_element_type=jnp.float32)


def _mid(p, b1, gamma, beta, w2s, w2d):
    return pl.pallas_call(
        _mid_body,
        out_shape=(jax.ShapeDtypeStruct((N, H2), jnp.float32),
                   jax.ShapeDtypeStruct((N, H2), jnp.float32)),
    )(p, b1, gamma, beta, w2s, w2d)


def _final_body(p_ref, b_ref, o_ref):
    o_ref[...] = p_ref[0] + p_ref[1] + b_ref[...]


def _final(p, b2):
    return pl.pallas_call(
        _final_body,
        out_shape=jax.ShapeDtypeStruct((N, H2), jnp.float32),
    )(p, b2)


_score1 = _make_score_kernel(H1)
_aggr1 = _make_aggr_kernel(H1)
_score2 = _make_score_kernel(H2)
_aggr2 = _make_aggr_kernel(H2)


def kernel(x, edge_index, W1s, W1d, a1, b1, gamma, beta, W2s, W2d, a2, b2):
    src = edge_index[0]
    dst = edge_index[1]

    xs1, xd1 = _proj(x, W1s, W1d, H1)
    ex1, den1 = _score1(xs1, xd1, src, dst, a1)
    p1 = _aggr1(xs1, src, dst, ex1, den1)
    xs2, xd2 = _mid(p1, b1, gamma, beta, W2s, W2d)
    ex2, den2 = _score2(xs2, xd2, src, dst, a2)
    p2 = _aggr2(xs2, src, dst, ex2, den2)
    return _final(p2, b2)


# speedup vs baseline: 8.2702x; 2.1187x over previous
import jax
import jax.numpy as jnp
from jax import lax
from jax.experimental import pallas as pl
from jax.experimental.pallas import tpu as pltpu
from jax.experimental.pallas import tpu_sc as plsc

N = 10000
E = 320000
D = 128
H1 = 64
H2 = 32

NC = 2
NS = 16
NW = NC * NS
L = 16

EPW = E // NW
CH = 80
NCHUNK = EPW // CH
RPT = 624
TAIL = N - NS * RPT


def _mesh():
    return plsc.VectorSubcoreMesh(core_axis_name="c", subcore_axis_name="s")


def _make_score_kernel(H):

    def body(xs_hbm, xd_hbm, src_hbm, dst_hbm, a_hbm,
             ex_hbm, denomp_hbm,
             xs_sp, xd_sp, denom_sp,
             src_v, dst_v, gx_v, gd_v, ex_v, a_v, zero_v, sem):
        c = lax.axis_index("c")
        s = lax.axis_index("s")
        wid = s * NC + c

        r0 = s * RPT
        pltpu.sync_copy(xs_hbm.at[pl.ds(r0, RPT)], xs_sp.at[pl.ds(r0, RPT)])
        pltpu.sync_copy(xd_hbm.at[pl.ds(r0, RPT)], xd_sp.at[pl.ds(r0, RPT)])

        def zb(i, _):
            zero_v[pl.ds(i * L, L)] = jnp.zeros((L,), jnp.float32)
            return 0
        lax.fori_loop(0, RPT // L, zb, 0, unroll=8)
        pltpu.sync_copy(zero_v, denom_sp.at[pl.ds(r0, RPT)])

        @pl.when(s == NS - 1)
        def _():
            pltpu.sync_copy(xs_hbm.at[pl.ds(NS * RPT, TAIL)],
                            xs_sp.at[pl.ds(NS * RPT, TAIL)])
            pltpu.sync_copy(xd_hbm.at[pl.ds(NS * RPT, TAIL)],
                            xd_sp.at[pl.ds(NS * RPT, TAIL)])
            pltpu.sync_copy(zero_v.at[pl.ds(0, TAIL)],
                            denom_sp.at[pl.ds(NS * RPT, TAIL)])

        pltpu.sync_copy(a_hbm, a_v)
        plsc.subcore_barrier()

        lanes = lax.iota(jnp.int32, L)
        lane15 = lanes == (L - 1)
        a_regs = [a_v[pl.ds(j * L, L)] for j in range(H // L)]

        def chunk_body(ci, _):
            base = wid * EPW + ci * CH
            pltpu.sync_copy(src_hbm.at[pl.ds(base, CH)], src_v)
            pltpu.sync_copy(dst_hbm.at[pl.ds(base, CH)], dst_v)
            pltpu.async_copy(xs_sp.at[src_v], gx_v, sem).wait()
            pltpu.async_copy(xd_sp.at[dst_v], gd_v, sem).wait()

            def edge_body(k, _):
                acc = jnp.zeros((L,), jnp.float32)
                for j in range(H // L):
                    m = (gx_v[k, pl.ds(j * L, L)]
                         + gd_v[k, pl.ds(j * L, L)])
                    m = jnp.maximum(m, 0.2 * m)
                    acc = acc + m * a_regs[j]
                sc = plsc.cumsum(acc)
                plsc.store_scatter(ex_v, [jnp.full((L,), 0, jnp.int32) + k],
                                   sc, mask=lane15)
                return 0
            lax.fori_loop(0, CH, edge_body, 0, unroll=8)
            for g in range(CH // L):
                ex_v[pl.ds(g * L, L)] = jnp.exp(ex_v[pl.ds(g * L, L)])

            pltpu.sync_copy(ex_v, ex_hbm.at[pl.ds(base, CH)])
            pltpu.sync_copy(ex_v, denom_sp.at[dst_v], add=True)
            return 0
        lax.fori_loop(0, NCHUNK, chunk_body, 0)

        plsc.subcore_barrier()
        pltpu.sync_copy(denom_sp.at[pl.ds(r0, RPT)],
                        denomp_hbm.at[pl.ds(c * N + r0, RPT)])

        @pl.when(s == NS - 1)
        def _():
            pltpu.sync_copy(denom_sp.at[pl.ds(NS * RPT, TAIL)],
                            denomp_hbm.at[pl.ds(c * N + NS * RPT, TAIL)])

    return pl.kernel(
        body,
        out_type=(jax.ShapeDtypeStruct((E,), jnp.float32),
                  jax.ShapeDtypeStruct((NC * N,), jnp.float32)),
        mesh=_mesh(),
        compiler_params=pltpu.CompilerParams(use_tc_tiling_on_sc=False, needs_layout_passes=False),
        scratch_types=[
            pltpu.VMEM_SHARED((N, H), jnp.float32),
            pltpu.VMEM_SHARED((N, H), jnp.float32),
            pltpu.VMEM_SHARED((N,), jnp.float32),
            pltpu.VMEM((CH,), jnp.int32),
            pltpu.VMEM((CH,), jnp.int32),
            pltpu.VMEM((CH, H), jnp.float32),
            pltpu.VMEM((CH, H), jnp.float32),
            pltpu.VMEM((CH,), jnp.float32),
            pltpu.VMEM((H,), jnp.float32),
            pltpu.VMEM((RPT,), jnp.float32),
            pltpu.SemaphoreType.DMA,
        ],
    )


def _make_aggr_kernel(H):

    def body(xs_hbm, src_hbm, dst_hbm, ex_hbm, denomp_hbm,
             outp_hbm,
             xs_sp, out_sp,
             src_v, dst_v, gx_v, wb_v, ex_v, al_v, d0_v, d1_v, zrow_v, sem):
        c = lax.axis_index("c")
        s = lax.axis_index("s")
        wid = s * NC + c

        r0 = s * RPT
        pltpu.sync_copy(xs_hbm.at[pl.ds(r0, RPT)], xs_sp.at[pl.ds(r0, RPT)])

        def zb(i, _):
            for j in range(H // L):
                zrow_v[i, pl.ds(j * L, L)] = jnp.zeros((L,), jnp.float32)
            return 0
        lax.fori_loop(0, RPT // 3, zb, 0, unroll=8)
        for q in range(3):
            pltpu.sync_copy(zrow_v, out_sp.at[pl.ds(r0 + q * (RPT // 3),
                                                    RPT // 3)])

        @pl.when(s == NS - 1)
        def _():
            pltpu.sync_copy(xs_hbm.at[pl.ds(NS * RPT, TAIL)],
                            xs_sp.at[pl.ds(NS * RPT, TAIL)])
            pltpu.sync_copy(zrow_v.at[pl.ds(0, TAIL)],
                            out_sp.at[pl.ds(NS * RPT, TAIL)])

        pltpu.sync_copy(denomp_hbm.at[pl.ds(0, N)], d0_v)
        pltpu.sync_copy(denomp_hbm.at[pl.ds(N, N)], d1_v)

        def addb(i, _):
            d0_v[pl.ds(i * L, L)] = (d0_v[pl.ds(i * L, L)]
                                     + d1_v[pl.ds(i * L, L)] + 1e-16)
            return 0
        lax.fori_loop(0, N // L, addb, 0, unroll=8)
        plsc.subcore_barrier()

        lanes = lax.iota(jnp.int32, L)

        def chunk_body(ci, _):
            base = wid * EPW + ci * CH
            pltpu.sync_copy(src_hbm.at[pl.ds(base, CH)], src_v)
            pltpu.sync_copy(dst_hbm.at[pl.ds(base, CH)], dst_v)
            pltpu.sync_copy(ex_hbm.at[pl.ds(base, CH)], ex_v)
            pltpu.async_copy(xs_sp.at[src_v], gx_v, sem).wait()

            for g in range(CH // L):
                didx = dst_v[pl.ds(g * L, L)]
                dv = plsc.load_gather(d0_v, [didx])
                al_v[pl.ds(g * L, L)] = ex_v[pl.ds(g * L, L)] / dv

            def edge_body(k, _):
                ab = plsc.load_gather(al_v,
                                      [jnp.full((L,), 0, jnp.int32) + k])
                for j in range(H // L):
                    wb_v[k, pl.ds(j * L, L)] = gx_v[k, pl.ds(j * L, L)] * ab
                return 0
            lax.fori_loop(0, CH, edge_body, 0, unroll=8)

            pltpu.sync_copy(wb_v, out_sp.at[dst_v], add=True)
            return 0
        lax.fori_loop(0, NCHUNK, chunk_body, 0)

        plsc.subcore_barrier()
        pltpu.sync_copy(out_sp.at[pl.ds(r0, RPT)],
                        outp_hbm.at[c, pl.ds(r0, RPT)])

        @pl.when(s == NS - 1)
        def _():
            pltpu.sync_copy(out_sp.at[pl.ds(NS * RPT, TAIL)],
                            outp_hbm.at[c, pl.ds(NS * RPT, TAIL)])

    return pl.kernel(
        body,
        out_type=jax.ShapeDtypeStruct((NC, N, H), jnp.float32),
        mesh=_mesh(),
        compiler_params=pltpu.CompilerParams(use_tc_tiling_on_sc=False, needs_layout_passes=False),
        scratch_types=[
            pltpu.VMEM_SHARED((N, H), jnp.float32),
            pltpu.VMEM_SHARED((N, H), jnp.float32),
            pltpu.VMEM((CH,), jnp.int32),
            pltpu.VMEM((CH,), jnp.int32),
            pltpu.VMEM((CH, H), jnp.float32),
            pltpu.VMEM((CH, H), jnp.float32),
            pltpu.VMEM((CH,), jnp.float32),
            pltpu.VMEM((CH,), jnp.float32),
            pltpu.VMEM((N,), jnp.float32),
            pltpu.VMEM((N,), jnp.float32),
            pltpu.VMEM((RPT // 3, H), jnp.float32),
            pltpu.SemaphoreType.DMA,
        ],
    )


def _proj_body(x_ref, ws_ref, wd_ref, xs_ref, xd_ref):
    x = x_ref[...]
    xs_ref[...] = jnp.dot(x, ws_ref[...], preferred_element_type=jnp.float32)
    xd_ref[...] = jnp.dot(x, wd_ref[...], preferred_element_type=jnp.float32)


def _proj(x, ws, wd, h):
    return pl.pallas_call(
        _proj_body,
        out_shape=(jax.ShapeDtypeStruct((N, h), jnp.float32),
                   jax.ShapeDtypeStruct((N, h), jnp.float32)),
    )(x, ws, wd)


def _mid_body(p_ref, b_ref, g_ref, be_ref, ws_ref, wd_ref, xs_ref, xd_ref):
    h = p_ref[0] + p_ref[1] + b_ref[...]
    mean = jnp.mean(h, axis=0)
    var = jnp.mean((h - mean) ** 2, axis=0)
    h = (h - mean) / jnp.sqrt(var + 1e-5) * g_ref[...] + be_ref[...]
    h = jnp.maximum(h, 0.0)
    xs_ref[...] = jnp.dot(h, ws_ref[...], preferred_element_type=jnp.float32)
    xd_ref[...] = jnp.dot(h, wd_ref[...], preferred_element_type=jnp.float32)


def _mid(p, b1, gamma, beta, w2s, w2d):
    return pl.pallas_call(
        _mid_body,
        out_shape=(jax.ShapeDtypeStruct((N, H2), jnp.float32),
                   jax.ShapeDtypeStruct((N, H2), jnp.float32)),
    )(p, b1, gamma, beta, w2s, w2d)


def _final_body(p_ref, b_ref, o_ref):
    o_ref[...] = p_ref[0] + p_ref[1] + b_ref[...]


def _final(p, b2):
    return pl.pallas_call(
        _final_body,
        out_shape=jax.ShapeDtypeStruct((N, H2), jnp.float32),
    )(p, b2)


_score1 = _make_score_kernel(H1)
_aggr1 = _make_aggr_kernel(H1)
_score2 = _make_score_kernel(H2)
_aggr2 = _make_aggr_kernel(H2)


def kernel(x, edge_index, W1s, W1d, a1, b1, gamma, beta, W2s, W2d, a2, b2):
    src = edge_index[0]
    dst = edge_index[1]

    xs1, xd1 = _proj(x, W1s, W1d, H1)
    ex1, den1 = _score1(xs1, xd1, src, dst, a1)
    p1 = _aggr1(xs1, src, dst, ex1, den1)
    xs2, xd2 = _mid(p1, b1, gamma, beta, W2s, W2d)
    ex2, den2 = _score2(xs2, xd2, src, dst, a2)
    p2 = _aggr2(xs2, src, dst, ex2, den2)
    return _final(p2, b2)
